# baseline (speedup 1.0000x reference)
import dataclasses
import functools
import math

import jax
import jax.numpy as jnp
from jax import lax
from jax.experimental import pallas as pl
from jax.experimental.pallas import tpu as pltpu
from jax.experimental.pallas import tpu_sc as plsc

N_LEVELS = 16
F = 2
LOG2_HASHMAP = 19
BASE_RES = 16
PER_LEVEL_SCALE = 1.5
DIM = 3
HASHMAP_SIZE = 2 ** LOG2_HASHMAP
N_POINTS = 262144
MASK = HASHMAP_SIZE - 1
P1 = int(2654435761 - (1 << 32))
P2 = int(805459861)


def _levels():
    metas = []
    offset = 0
    for l in range(N_LEVELS):
        scale = BASE_RES * (PER_LEVEL_SCALE ** l) - 1.0
        res = int(math.ceil(scale)) + 1
        dense = (res + 1) ** DIM
        hashed = dense > HASHMAP_SIZE
        size = HASHMAP_SIZE if hashed else dense
        metas.append((scale, res, offset, size, hashed))
        offset += size
    return metas, offset

LEVELS, TOTAL_ROWS = _levels()

NW = 32
PPW = N_POINTS // NW
C = 256
NCH = PPW // C
L = 16


def _encode_kernel(xx, yy, zz, table, out, cxr, cyr, czr, wb, ib, fb, ob, sem):
    wid = lax.axis_index("s") * 2 + lax.axis_index("c")
    iot = lax.iota(jnp.int32, L)

    @pl.loop(0, NCH)
    def _chunk(ch):
        pbase = wid * PPW + ch * C
        pltpu.sync_copy(xx.at[pl.ds(pbase, C)], cxr)
        pltpu.sync_copy(yy.at[pl.ds(pbase, C)], cyr)
        pltpu.sync_copy(zz.at[pl.ds(pbase, C)], czr)

        for l, (scale, res, off, size, hashed) in enumerate(LEVELS):
            @pl.loop(0, C, step=L)
            def _grp(c0, l=l, scale=scale, res=res, off=off, hashed=hashed):
                px = cxr[pl.ds(c0, L)] * scale + 0.5
                py = cyr[pl.ds(c0, L)] * scale + 0.5
                pz = czr[pl.ds(c0, L)] * scale + 0.5
                bx = px.astype(jnp.int32)
                by = py.astype(jnp.int32)
                bz = pz.astype(jnp.int32)
                fx = px - bx.astype(jnp.float32)
                fy = py - by.astype(jnp.float32)
                fz = pz - bz.astype(jnp.float32)
                wx = (1.0 - fx, fx)
                wy = (1.0 - fy, fy)
                wz = (1.0 - fz, fz)
                if hashed:
                    hx = (bx, bx + 1)
                    t1 = by * P1
                    hy = (t1, t1 + P1)
                    t2 = bz * P2
                    hz = (t2, t2 + P2)
                else:
                    r1 = res + 1
                    hx = (bx + off, bx + off + 1)
                    t1 = by * r1
                    hy = (t1, t1 + r1)
                    t2 = bz * (r1 * r1)
                    hz = (t2, t2 + r1 * r1)
                for c in range(8):
                    i, j, k = c & 1, (c >> 1) & 1, (c >> 2) & 1
                    if hashed:
                        idx = ((hx[i] ^ hy[j] ^ hz[k]) & MASK) + off
                    else:
                        idx = hx[i] + hy[j] + hz[k]
                    ib[pl.ds(c * C + c0, L)] = idx
                    wb[c, pl.ds(c0, L)] = wx[i] * wy[j] * wz[k]

            pltpu.sync_copy(table.at[ib], fb)

            @pl.loop(0, C, step=L)
            def _blend(c0, l=l):
                col0 = jnp.full((L,), 0, jnp.int32)
                col1 = jnp.full((L,), 1, jnp.int32)
                acc0 = jnp.zeros((L,), jnp.float32)
                acc1 = jnp.zeros((L,), jnp.float32)
                for c in range(8):
                    w = wb[c, pl.ds(c0, L)]
                    ridx = iot + (c * C) + c0
                    f0 = plsc.load_gather(fb, [ridx, col0])
                    f1 = plsc.load_gather(fb, [ridx, col1])
                    acc0 = acc0 + w * f0
                    acc1 = acc1 + w * f1
                rows = iot + c0
                plsc.store_scatter(ob, [rows, jnp.full((L,), 2 * l, jnp.int32)], acc0)
                plsc.store_scatter(ob, [rows, jnp.full((L,), 2 * l + 1, jnp.int32)], acc1)

        pltpu.sync_copy(ob, out.at[pl.ds(pbase, C)])


@jax.jit
def kernel(x, table):
    xt = x.T
    xx, yy, zz = xt[0], xt[1], xt[2]
    tab8 = jnp.pad(table, ((0, 0), (0, 8 - F)))
    mesh = plsc.VectorSubcoreMesh(core_axis_name="c", subcore_axis_name="s")
    cp = pltpu.CompilerParams(
        needs_layout_passes=False,
        use_tc_tiling_on_sc=False,
    )
    run = pl.kernel(
        _encode_kernel,
        out_type=jax.ShapeDtypeStruct((N_POINTS, 2 * N_LEVELS), jnp.float32),
        mesh=mesh,
        scratch_types=[
            pltpu.VMEM((C,), jnp.float32),
            pltpu.VMEM((C,), jnp.float32),
            pltpu.VMEM((C,), jnp.float32),
            pltpu.VMEM((8, C), jnp.float32),
            pltpu.VMEM((8 * C,), jnp.int32),
            pltpu.VMEM((8 * C, 8), jnp.float32),
            pltpu.VMEM((C, 2 * N_LEVELS), jnp.float32),
            pltpu.SemaphoreType.DMA,
        ],
        compiler_params=cp,
    )
    return run(xx, yy, zz, tab8)
